# baseline (speedup 1.0000x reference)
import functools

import jax
import jax.numpy as jnp
import numpy as np
from jax import lax
from jax.experimental import pallas as pl
from jax.experimental.pallas import tpu as pltpu

_K0 = np.uint32(0)
_K1 = np.uint32(42)
_KS2 = np.uint32(_K0 ^ _K1 ^ np.uint32(0x1BD11BDA))
_ROT = (13, 15, 26, 6, 17, 29, 16, 24, 13, 15, 26, 6, 17, 29, 16, 24, 13, 15, 26, 6)
_INJ = ((1, 2, 1), (2, 0, 2), (0, 1, 3), (1, 2, 4), (2, 0, 5))
_TINY = np.float32(np.finfo(np.float32).tiny)


def _threefry_bits(linear_idx_u32):
    ks = (_K0, _K1, _KS2)
    x0 = jnp.full(linear_idx_u32.shape, _K0, dtype=jnp.uint32)
    x1 = linear_idx_u32 + _K1
    for chunk, (a, b, c) in enumerate(_INJ):
        for r in _ROT[4 * chunk:4 * chunk + 4]:
            x0 = x0 + x1
            x1 = (x1 << np.uint32(r)) | (x1 >> np.uint32(32 - r))
            x1 = x1 ^ x0
        x0 = x0 + ks[a]
        x1 = x1 + ks[b] + np.uint32(c)
    return x0 ^ x1


_INT_MAX = np.int32(np.iinfo(np.int32).max)
_NEG_INF = np.float32(-np.inf)


def _row_kernel(x_ref, probs_ref, samp_ref, *, ncols, nsub, cw, clane, nlc, ntail):
    row = pl.program_id(0)

    m = jnp.max(x_ref[...])

    def step(off, cl, carry):
        s, bv, bi = carry
        xc = x_ref[:, :, pl.ds(off, cl)]
        e = jnp.exp(xc - m)
        probs_ref[:, :, pl.ds(off, cl)] = e
        s = s + jnp.sum(e)

        subl = lax.broadcasted_iota(jnp.int32, (1, nsub, cl), 1) * cw
        lane = lax.broadcasted_iota(jnp.int32, (1, nsub, cl), 2)
        icol = subl + lane + off

        idx = icol.astype(jnp.uint32) + jnp.uint32(row * ncols)
        bits = _threefry_bits(idx)
        fb = (bits >> np.uint32(9)) | np.uint32(0x3F800000)
        f = lax.bitcast_convert_type(fb, jnp.float32) - np.float32(1.0)
        u = jnp.maximum(_TINY, f * np.float32(1.0 - _TINY) + _TINY)
        g = -jnp.log(-jnp.log(u))

        v = xc + g
        vm = jnp.max(v)
        vi = jnp.min(jnp.where(v == vm, icol, _INT_MAX))
        upd = (vm > bv) | ((vm == bv) & (vi < bi))
        bv = jnp.where(upd, vm, bv)
        bi = jnp.where(upd, vi, bi)
        return s, bv, bi

    carry = (jnp.float32(0.0), _NEG_INF, _INT_MAX)
    for k in range(nlc):
        carry = step(k * clane, clane, carry)
    if ntail:
        carry = step(nlc * clane, ntail, carry)
    s, _, bi = carry

    probs_ref[...] = probs_ref[...] * (np.float32(1.0) / s)
    samp_ref[...] = jnp.full(samp_ref.shape, bi, dtype=jnp.int32)


def kernel(logits):
    nrows, ncols = logits.shape
    nsub = 8 if ncols % 8 == 0 else 1
    cw = ncols // nsub
    nfull = cw // 128
    clane = min(128, max(nfull, 1) * 128)
    nlc = (nfull * 128) // clane
    ntail = cw - nlc * clane
    logits3 = logits.reshape(nrows, nsub, cw)
    nchunks = nsub
    probs3, samples3 = pl.pallas_call(
        functools.partial(_row_kernel, ncols=ncols, nsub=nsub, cw=cw,
                          clane=clane, nlc=nlc, ntail=ntail),
        grid=(nrows,),
        in_specs=[pl.BlockSpec((1, nchunks, cw), lambda r: (r, 0, 0))],
        out_specs=[
            pl.BlockSpec((1, nchunks, cw), lambda r: (r, 0, 0)),
            pl.BlockSpec((1, 1, 128), lambda r: (r, 0, 0)),
        ],
        out_shape=[
            jax.ShapeDtypeStruct((nrows, nchunks, cw), jnp.float32),
            jax.ShapeDtypeStruct((nrows, 1, 128), jnp.int32),
        ],
        compiler_params=pltpu.CompilerParams(
            dimension_semantics=("arbitrary",),
        ),
    )(logits3)
    samples = samples3[:, 0, 0]
    probs = probs3.reshape(nrows, ncols)
    return (samples, probs)


# speedup vs baseline: 12.3306x; 12.3306x over previous
import functools

import jax
import jax.numpy as jnp
import numpy as np
from jax import lax
from jax.experimental import pallas as pl
from jax.experimental.pallas import tpu as pltpu

_K0 = np.uint32(0)
_K1 = np.uint32(42)
_KS2 = np.uint32(_K0 ^ _K1 ^ np.uint32(0x1BD11BDA))
_ROT = (13, 15, 26, 6, 17, 29, 16, 24, 13, 15, 26, 6, 17, 29, 16, 24, 13, 15, 26, 6)
_INJ = ((1, 2, 1), (2, 0, 2), (0, 1, 3), (1, 2, 4), (2, 0, 5))
_TINY = np.float32(np.finfo(np.float32).tiny)
_INT_MAX = np.int32(np.iinfo(np.int32).max)
_NEG_INF = np.float32(-np.inf)


@functools.lru_cache(maxsize=4)
def _host_uniform(nrows, ncols):
    n = nrows * ncols
    i = np.arange(n, dtype=np.uint64)
    x0 = (i >> np.uint64(32)).astype(np.uint32)
    x1 = i.astype(np.uint32)
    ks = (_K0, _K1, _KS2)
    x0 = (x0 + ks[0]).astype(np.uint32)
    x1 = (x1 + ks[1]).astype(np.uint32)
    for chunk, (a, b, c) in enumerate(_INJ):
        for r in _ROT[4 * chunk:4 * chunk + 4]:
            x0 += x1
            x1 = (x1 << np.uint32(r)) | (x1 >> np.uint32(32 - r))
            x1 ^= x0
        x0 = (x0 + ks[a]).astype(np.uint32)
        x1 = (x1 + ks[b] + np.uint32(c)).astype(np.uint32)
    bits = x0 ^ x1
    del x0, x1, i
    fb = (bits >> np.uint32(9)) | np.uint32(0x3F800000)
    f = fb.view(np.float32) - np.float32(1.0)
    u = np.maximum(_TINY, f * np.float32(1.0 - _TINY) + _TINY)
    return u.reshape(nrows, ncols)


def _row_kernel(x_ref, u_ref, probs_ref, samp_ref, *, nsub, cw, clane, nlc, ntail):
    m = jnp.max(x_ref[...])

    def step(off, cl, carry):
        s, bv, bi = carry
        xc = x_ref[:, :, pl.ds(off, cl)]
        e = jnp.exp(xc - m)
        probs_ref[:, :, pl.ds(off, cl)] = e
        s = s + jnp.sum(e)

        uc = u_ref[:, :, pl.ds(off, cl)]
        g = -jnp.log(-jnp.log(uc))
        v = xc + g
        vm = jnp.max(v)
        subl = lax.broadcasted_iota(jnp.int32, (1, nsub, cl), 1) * cw
        lane = lax.broadcasted_iota(jnp.int32, (1, nsub, cl), 2)
        icol = subl + lane + off
        vi = jnp.min(jnp.where(v == vm, icol, _INT_MAX))
        upd = (vm > bv) | ((vm == bv) & (vi < bi))
        bv = jnp.where(upd, vm, bv)
        bi = jnp.where(upd, vi, bi)
        return s, bv, bi

    carry = (jnp.float32(0.0), _NEG_INF, _INT_MAX)
    for k in range(nlc):
        carry = step(k * clane, clane, carry)
    if ntail:
        carry = step(nlc * clane, ntail, carry)
    s, _, bi = carry

    probs_ref[...] = probs_ref[...] * (np.float32(1.0) / s)
    samp_ref[...] = jnp.full(samp_ref.shape, bi, dtype=jnp.int32)


def kernel(logits):
    nrows, ncols = logits.shape
    nsub = 8 if ncols % 8 == 0 else 1
    cw = ncols // nsub
    nfull = cw // 128
    clane = max(nfull // 8, 1) * 128
    nlc = (nfull * 128) // clane
    ntail = cw - nlc * clane
    logits3 = logits.reshape(nrows, nsub, cw)
    u3 = jnp.asarray(_host_uniform(nrows, ncols).reshape(nrows, nsub, cw))
    probs3, samples3 = pl.pallas_call(
        functools.partial(_row_kernel, nsub=nsub, cw=cw,
                          clane=clane, nlc=nlc, ntail=ntail),
        grid=(nrows,),
        in_specs=[
            pl.BlockSpec((1, nsub, cw), lambda r: (r, 0, 0)),
            pl.BlockSpec((1, nsub, cw), lambda r: (r, 0, 0)),
        ],
        out_specs=[
            pl.BlockSpec((1, nsub, cw), lambda r: (r, 0, 0)),
            pl.BlockSpec((1, 1, 128), lambda r: (r, 0, 0)),
        ],
        out_shape=[
            jax.ShapeDtypeStruct((nrows, nsub, cw), jnp.float32),
            jax.ShapeDtypeStruct((nrows, 1, 128), jnp.int32),
        ],
        compiler_params=pltpu.CompilerParams(
            dimension_semantics=("arbitrary",),
        ),
    )(logits3, u3)
    samples = samples3[:, 0, 0]
    probs = probs3.reshape(nrows, ncols)
    return (samples, probs)


# speedup vs baseline: 27.2626x; 2.2110x over previous
import functools

import jax
import jax.numpy as jnp
import numpy as np
from jax import lax
from jax.experimental import pallas as pl
from jax.experimental.pallas import tpu as pltpu

_K0 = np.uint32(0)
_K1 = np.uint32(42)
_KS2 = np.uint32(_K0 ^ _K1 ^ np.uint32(0x1BD11BDA))
_ROT = (13, 15, 26, 6, 17, 29, 16, 24, 13, 15, 26, 6, 17, 29, 16, 24, 13, 15, 26, 6)
_INJ = ((1, 2, 1), (2, 0, 2), (0, 1, 3), (1, 2, 4), (2, 0, 5))
_TINY = np.float32(np.finfo(np.float32).tiny)
_INT_MAX = np.int32(np.iinfo(np.int32).max)
_NEG_INF = np.float32(-np.inf)


@functools.lru_cache(maxsize=4)
def _host_uniform(nrows, ncols, ncols_pad):
    n = nrows * ncols
    i = np.arange(n, dtype=np.uint64)
    x0 = (i >> np.uint64(32)).astype(np.uint32)
    x1 = i.astype(np.uint32)
    ks = (_K0, _K1, _KS2)
    x0 = (x0 + ks[0]).astype(np.uint32)
    x1 = (x1 + ks[1]).astype(np.uint32)
    for chunk, (a, b, c) in enumerate(_INJ):
        for r in _ROT[4 * chunk:4 * chunk + 4]:
            x0 += x1
            x1 = (x1 << np.uint32(r)) | (x1 >> np.uint32(32 - r))
            x1 ^= x0
        x0 = (x0 + ks[a]).astype(np.uint32)
        x1 = (x1 + ks[b] + np.uint32(c)).astype(np.uint32)
    bits = x0 ^ x1
    del x0, x1, i
    fb = (bits >> np.uint32(9)) | np.uint32(0x3F800000)
    f = fb.view(np.float32) - np.float32(1.0)
    u = np.maximum(_TINY, f * np.float32(1.0 - _TINY) + _TINY)
    u = u.reshape(nrows, ncols)
    if ncols_pad > ncols:
        u = np.pad(u, ((0, 0), (0, ncols_pad - ncols)),
                   constant_values=_TINY)
    return u


def _stats_kernel(x_ref, u_ref, m_out, s_out, samp_out,
                  m_sc, s_sc, bv_sc, bi_sc, *, ncols, cwidth, nlc):
    c = pl.program_id(1)
    x = x_ref[...]
    lane = lax.broadcasted_iota(jnp.int32, x.shape, 1)
    icol = lane + c * cwidth
    valid = icol < ncols
    xm = jnp.where(valid, x, _NEG_INF)
    cm = jnp.max(xm, axis=1, keepdims=True)

    g = -jnp.log(-jnp.log(u_ref[...]))
    v = jnp.where(valid, x + g, _NEG_INF)
    vm = jnp.max(v, axis=1, keepdims=True)
    vi = jnp.min(jnp.where(v == vm, icol, _INT_MAX), axis=1, keepdims=True)

    @pl.when(c == 0)
    def _init():
        m_sc[...] = jnp.broadcast_to(cm, m_sc.shape)
        s0 = jnp.sum(jnp.exp(xm - cm), axis=1, keepdims=True)
        s_sc[...] = jnp.broadcast_to(s0, s_sc.shape)
        bv_sc[...] = jnp.broadcast_to(vm, bv_sc.shape)
        bi_sc[...] = jnp.broadcast_to(vi, bi_sc.shape)

    @pl.when(c > 0)
    def _accum():
        m_old = m_sc[:, :1]
        m_new = jnp.maximum(m_old, cm)
        s_new = (s_sc[:, :1] * jnp.exp(m_old - m_new)
                 + jnp.sum(jnp.exp(xm - m_new), axis=1, keepdims=True))
        m_sc[...] = jnp.broadcast_to(m_new, m_sc.shape)
        s_sc[...] = jnp.broadcast_to(s_new, s_sc.shape)
        bv_old = bv_sc[:, :1]
        bi_old = bi_sc[:, :1]
        upd = (vm > bv_old) | ((vm == bv_old) & (vi < bi_old))
        bv_sc[...] = jnp.broadcast_to(jnp.where(upd, vm, bv_old), bv_sc.shape)
        bi_sc[...] = jnp.broadcast_to(jnp.where(upd, vi, bi_old), bi_sc.shape)

    @pl.when(c == nlc - 1)
    def _emit():
        m_out[...] = m_sc[...]
        s_out[...] = s_sc[...]
        samp_out[...] = bi_sc[...]


def _norm_kernel(x_ref, m_ref, s_ref, probs_ref):
    m = m_ref[:, :1]
    inv = np.float32(1.0) / s_ref[:, :1]
    probs_ref[...] = jnp.exp(x_ref[...] - m) * inv


def kernel(logits):
    nrows, ncols = logits.shape
    grows = 8 if nrows % 8 == 0 else nrows
    cwidth = min(131072, -(-ncols // 128) * 128)
    nlc = -(-ncols // cwidth)
    ngroups = nrows // grows
    u_pad = jnp.asarray(_host_uniform(nrows, ncols, nlc * cwidth))

    m, s, samp = pl.pallas_call(
        functools.partial(_stats_kernel, ncols=ncols, cwidth=cwidth, nlc=nlc),
        grid=(ngroups, nlc),
        in_specs=[
            pl.BlockSpec((grows, cwidth), lambda g, c: (g, c)),
            pl.BlockSpec((grows, cwidth), lambda g, c: (g, c)),
        ],
        out_specs=[
            pl.BlockSpec((grows, 128), lambda g, c: (g, 0)),
            pl.BlockSpec((grows, 128), lambda g, c: (g, 0)),
            pl.BlockSpec((grows, 128), lambda g, c: (g, 0)),
        ],
        out_shape=[
            jax.ShapeDtypeStruct((nrows, 128), jnp.float32),
            jax.ShapeDtypeStruct((nrows, 128), jnp.float32),
            jax.ShapeDtypeStruct((nrows, 128), jnp.int32),
        ],
        scratch_shapes=[
            pltpu.VMEM((grows, 128), jnp.float32),
            pltpu.VMEM((grows, 128), jnp.float32),
            pltpu.VMEM((grows, 128), jnp.float32),
            pltpu.VMEM((grows, 128), jnp.int32),
        ],
        compiler_params=pltpu.CompilerParams(
            dimension_semantics=("arbitrary", "arbitrary"),
        ),
    )(logits, u_pad)

    probs = pl.pallas_call(
        _norm_kernel,
        grid=(ngroups, nlc),
        in_specs=[
            pl.BlockSpec((grows, cwidth), lambda g, c: (g, c)),
            pl.BlockSpec((grows, 128), lambda g, c: (g, 0)),
            pl.BlockSpec((grows, 128), lambda g, c: (g, 0)),
        ],
        out_specs=pl.BlockSpec((grows, cwidth), lambda g, c: (g, c)),
        out_shape=jax.ShapeDtypeStruct((nrows, ncols), jnp.float32),
        compiler_params=pltpu.CompilerParams(
            dimension_semantics=("parallel", "arbitrary"),
        ),
    )(logits, m, s)

    samples = samp[:, 0]
    return (samples, probs)
